# baseline (speedup 1.0000x reference)
import functools

import jax
import jax.numpy as jnp
from jax import lax
from jax.experimental import pallas as pl
from jax.experimental.pallas import tpu as pltpu
from jax.experimental.pallas import tpu_sc as plsc

_B = 8
_N = 4096
_D = 64
_S = 512
_K = 32
_P = _B * _S * _K
_BS = _B * _S
_T = _P + _BS
_RB = 4096
_RBF = 128
_NBLK = _P // _RB
_EPS = 1e-5
_TW = 80


def _fps_body(xyz_ref, start_ref, cent_ref):
    x = xyz_ref[0]
    y = xyz_ref[1]
    z = xyz_ref[2]
    xyz24 = jnp.concatenate([x, y, z], axis=0)
    iota_n = lax.broadcasted_iota(jnp.int32, (_B, _N), 1)
    iota_n24 = lax.broadcasted_iota(jnp.int32, (3 * _B, _N), 1)
    iota_s = lax.broadcasted_iota(jnp.int32, (_B, _S), 1)
    far0 = start_ref[...]
    dist0 = jnp.full((_B, _N), 1e10, dtype=jnp.float32)
    cents0 = jnp.zeros((_B, _S), dtype=jnp.int32)

    def body(i, state):
        distance, far, cents = state
        cents = jnp.where(iota_s == i, far, cents)
        far24 = jnp.concatenate([far, far, far], axis=0)
        csum = jnp.sum(jnp.where(iota_n24 == far24, xyz24, 0.0),
                       axis=1, keepdims=True)
        dx = x - csum[0:_B]
        dy = y - csum[_B:2 * _B]
        dz = z - csum[2 * _B:3 * _B]
        d = (dx * dx + dy * dy) + dz * dz
        distance = jnp.minimum(distance, d)
        mx = jnp.max(distance, axis=1, keepdims=True)
        far = jnp.min(jnp.where(distance == mx, iota_n, _N), axis=1,
                      keepdims=True)
        return distance, far, cents

    _, _, cents = lax.fori_loop(0, _S, body, (dist0, far0, cents0))
    cent_ref[...] = cents


def _run_fps(xyz3, start):
    return pl.pallas_call(
        _fps_body,
        out_shape=jax.ShapeDtypeStruct((_B, _S), jnp.int32),
    )(xyz3, start)


_SB = 128


def _knn_body(xyz_ref, cent_ref, nxyz_ref, nbr_ref, cg_ref):
    b = pl.program_id(0)
    c = cent_ref[0]
    xb = xyz_ref[0, 0]
    yb = xyz_ref[1, 0]
    zb = xyz_ref[2, 0]
    iota_n = lax.broadcasted_iota(jnp.int32, (_SB, _N), 1)
    onehot = iota_n == c
    cx = jnp.sum(jnp.where(onehot, xb, 0.0), axis=1, keepdims=True)
    cy = jnp.sum(jnp.where(onehot, yb, 0.0), axis=1, keepdims=True)
    cz = jnp.sum(jnp.where(onehot, zb, 0.0), axis=1, keepdims=True)
    nxyz_ref[...] = jnp.concatenate([cx, cy, cz], axis=1)[None]
    dx = xb - cx
    dy = yb - cy
    dz = zb - cz
    d2 = (dx * dx + dy * dy) + dz * dz
    iota_k = lax.broadcasted_iota(jnp.int32, (_SB, _K), 1)
    key = (lax.bitcast_convert_type(d2, jnp.int32) & ~0xFFF) | iota_n
    big = jnp.int32(0x7FFFFFFF)

    def sel_body(k, state):
        keyc, sel = state
        m = jnp.min(keyc, axis=1, keepdims=True)
        keyc = jnp.where(keyc == m, big, keyc)
        sel = jnp.where(iota_k == (k - 1), m & 0xFFF, sel)
        return keyc, sel

    sel0 = jnp.zeros((_SB, _K), dtype=jnp.int32)
    _, sel = lax.fori_loop(0, _K + 1, sel_body, (key, sel0))
    base = b * _N
    nbr_ref[...] = (sel + base)[None]
    cg_ref[...] = (c + base)[None]


def _run_knn(xyz3, cents3):
    xyz4 = xyz3.reshape(3, _B, 1, _N)
    grid = (_B, _S // _SB)
    return pl.pallas_call(
        _knn_body,
        grid=grid,
        in_specs=[
            pl.BlockSpec((3, 1, 1, _N), lambda b, s: (0, b, 0, 0)),
            pl.BlockSpec((1, _SB, 1), lambda b, s: (b, s, 0)),
        ],
        out_specs=[
            pl.BlockSpec((1, _SB, 3), lambda b, s: (b, s, 0)),
            pl.BlockSpec((1, _SB, _K), lambda b, s: (b, s, 0)),
            pl.BlockSpec((1, _SB, 1), lambda b, s: (b, s, 0)),
        ],
        out_shape=[
            jax.ShapeDtypeStruct((_B, _S, 3), jnp.float32),
            jax.ShapeDtypeStruct((_B, _S, _K), jnp.int32),
            jax.ShapeDtypeStruct((_B, _S, 1), jnp.int32),
        ],
    )(xyz4, cents3)


def _tp_pts_body(p_ref, o_ref):
    eye = (lax.broadcasted_iota(jnp.int32, (_D, _D), 0)
           == lax.broadcasted_iota(jnp.int32, (_D, _D), 1)).astype(jnp.float32)
    o_ref[0] = lax.dot_general(p_ref[0], eye, (((0,), (0,)), ((), ())),
                               preferred_element_type=jnp.float32)


def _tp_xyz_body(x_ref, o_ref):
    eye = (lax.broadcasted_iota(jnp.int32, (3, 16), 0)
           == lax.broadcasted_iota(jnp.int32, (3, 16), 1)).astype(jnp.float32)
    o_ref[0] = lax.dot_general(x_ref[0], eye, (((0,), (0,)), ((), ())),
                               preferred_element_type=jnp.float32)


_NT = 2048


def _build_tables(xyz, points):
    pts_t = pl.pallas_call(
        _tp_pts_body,
        grid=(_B, _N // _NT),
        in_specs=[pl.BlockSpec((1, _D, _NT), lambda b, j: (b, 0, j))],
        out_specs=pl.BlockSpec((1, _NT, _D), lambda b, j: (b, j, 0)),
        out_shape=jax.ShapeDtypeStruct((_B, _N, _D), jnp.float32),
    )(points)
    xyz_t = pl.pallas_call(
        _tp_xyz_body,
        grid=(_B, _N // _NT),
        in_specs=[pl.BlockSpec((1, 3, _NT), lambda b, j: (b, 0, j))],
        out_specs=pl.BlockSpec((1, _NT, 16), lambda b, j: (b, j, 0)),
        out_shape=jax.ShapeDtypeStruct((_B, _N, 16), jnp.float32),
    )(xyz)
    return xyz_t.reshape(_B * _N, 16), pts_t.reshape(_B * _N, _D)


_NW = 32
_CH = 128
_WPP = _P // _NW
_NCHP = _WPP // _CH
_WPF = _BS // _NW


def _gather_rows(xyz_t, pts_t, gidx):
    mesh = plsc.VectorSubcoreMesh(core_axis_name="c", subcore_axis_name="s")

    @functools.partial(
        pl.kernel,
        mesh=mesh,
        compiler_params=pltpu.CompilerParams(use_tc_tiling_on_sc=False),
        out_type=[
            jax.ShapeDtypeStruct((_P, 16), jnp.float32),
            jax.ShapeDtypeStruct((_P, _D), jnp.float32),
            jax.ShapeDtypeStruct((_BS, _D), jnp.float32),
        ],
        scratch_types=[
            pltpu.VMEM((_WPP + _WPF,), jnp.int32),
            pltpu.VMEM((_CH, 16), jnp.float32),
            pltpu.VMEM((_CH, _D), jnp.float32),
            pltpu.SemaphoreType.DMA,
            pltpu.SemaphoreType.DMA,
        ],
    )
    def body(xyz_hbm, pts_hbm, gidx_hbm, gx_hbm, gp_hbm, fp_hbm,
             idx_v, xbuf, pbuf, sem1, sem2):
        wid = lax.axis_index("s") * 2 + lax.axis_index("c")
        basep = wid * _WPP
        basef = wid * _WPF
        pltpu.sync_copy(gidx_hbm.at[pl.ds(basep, _WPP)],
                        idx_v.at[pl.ds(0, _WPP)])
        pltpu.sync_copy(gidx_hbm.at[pl.ds(_P + basef, _WPF)],
                        idx_v.at[pl.ds(_WPP, _WPF)])

        def chunk(j, carry):
            cp = pltpu.async_copy(
                pts_hbm.at[idx_v.at[pl.ds(j * _CH, _CH)]], pbuf, sem1)
            cx = pltpu.async_copy(
                xyz_hbm.at[idx_v.at[pl.ds(j * _CH, _CH)]], xbuf, sem2)
            cp.wait()
            cx.wait()
            pltpu.sync_copy(pbuf, gp_hbm.at[pl.ds(basep + j * _CH, _CH)])
            pltpu.sync_copy(xbuf, gx_hbm.at[pl.ds(basep + j * _CH, _CH)])
            return carry

        lax.fori_loop(0, _NCHP, chunk, 0, unroll=False)
        pltpu.async_copy(
            pts_hbm.at[idx_v.at[pl.ds(_WPP, _WPF)]], pbuf, sem1).wait()
        pltpu.sync_copy(pbuf, fp_hbm.at[pl.ds(basef, _WPF)])

    return body(xyz_t, pts_t, gidx)


def _rep_mat():
    rows = lax.broadcasted_iota(jnp.int32, (_RB, _RBF), 0) // _K
    cols = lax.broadcasted_iota(jnp.int32, (_RB, _RBF), 1)
    return (rows == cols).astype(jnp.float32)


def _bn_relu(x, stat_ref, count):
    st = stat_ref[...]
    mean = st[0:1, :] / count
    var = st[1:2, :] / count - mean * mean
    return jnp.maximum((x - mean) * lax.rsqrt(var + _EPS), 0.0)


def _acc_stats(stat_ref, y):
    s = jnp.sum(y, axis=0, keepdims=True)
    s2 = jnp.sum(y * y, axis=0, keepdims=True)
    st = jnp.concatenate([s, s2, jnp.zeros((6, y.shape[1]), jnp.float32)],
                         axis=0)

    @pl.when(pl.program_id(0) == 0)
    def _():
        stat_ref[...] = st

    @pl.when(pl.program_id(0) != 0)
    def _():
        stat_ref[...] += st


def _dk1_body(gx_ref, nx_ref, pawt_ref, pab_ref, y_ref, stat_ref):
    gxyz = gx_ref[...][:, 0:3]
    cen = jnp.dot(_rep_mat(), nx_ref[...],
                  preferred_element_type=jnp.float32)
    d = gxyz - cen
    gnorm = d * d
    gdist = jnp.sqrt(jnp.sum(gnorm, axis=1, keepdims=True))
    feat = jnp.concatenate([cen, gxyz, gnorm, gdist], axis=1)
    y = jnp.dot(feat, pawt_ref[...],
                preferred_element_type=jnp.float32) + pab_ref[...]
    y_ref[...] = y
    _acc_stats(stat_ref, y)


def _dk2_body(y_ref, ystat_ref, gp_ref, fp_ref, w0pt_ref, w0at_ref, b0_ref,
              z0_ref, f0_ref, z0stat_ref, f0stat_ref):
    aug = _bn_relu(y_ref[...], ystat_ref, float(_P))
    z0 = (jnp.dot(gp_ref[...], w0pt_ref[...],
                  preferred_element_type=jnp.float32)
          + jnp.dot(aug, w0at_ref[...], preferred_element_type=jnp.float32)
          + b0_ref[...])
    z0_ref[...] = z0
    _acc_stats(z0stat_ref, z0)
    maxo = jnp.max(aug.reshape(_RBF, _K, 32), axis=1)
    f0 = (jnp.dot(fp_ref[...], w0pt_ref[...],
                  preferred_element_type=jnp.float32)
          + jnp.dot(maxo, w0at_ref[...], preferred_element_type=jnp.float32)
          + b0_ref[...])
    f0_ref[...] = f0
    _acc_stats(f0stat_ref, f0)


def _dk4_body(z0_ref, z0stat_ref, f0_ref, f0stat_ref, w1t_ref, b1_ref,
              z1_ref, f1_ref, z1stat_ref, f1stat_ref):
    z = _bn_relu(z0_ref[...], z0stat_ref, float(_P))
    z1 = jnp.dot(z, w1t_ref[...],
                 preferred_element_type=jnp.float32) + b1_ref[...]
    z1_ref[...] = z1
    _acc_stats(z1stat_ref, z1)
    f = _bn_relu(f0_ref[...], f0stat_ref, float(_BS))
    f1 = jnp.dot(f, w1t_ref[...],
                 preferred_element_type=jnp.float32) + b1_ref[...]
    f1_ref[...] = f1
    _acc_stats(f1stat_ref, f1)


def _dk5_body(z1_ref, z1stat_ref, f1_ref, f1stat_ref, gx_ref, nx_ref,
              lwt_ref, lb_ref, w_ref, wstat_ref):
    fpc = _bn_relu(z1_ref[...], z1stat_ref, float(_P))
    npc = _bn_relu(f1_ref[...], f1stat_ref, float(_BS))
    rep = _rep_mat()
    npc_rep = jnp.dot(rep, npc, preferred_element_type=jnp.float32)
    delta = fpc - npc_rep
    gxyz = gx_ref[...][:, 0:3]
    cen = jnp.dot(rep, nx_ref[...], preferred_element_type=jnp.float32)
    d = gxyz - cen
    gdist = jnp.sqrt(jnp.sum(d * d, axis=1, keepdims=True))
    feat = jnp.concatenate([cen, gxyz, gdist, delta], axis=1)
    w = jnp.dot(feat, lwt_ref[...],
                preferred_element_type=jnp.float32) + lb_ref[...]
    w_ref[...] = w
    _acc_stats(wstat_ref, w)


def _dk6_body(w_ref, wstat_ref, z1_ref, z1stat_ref, f1_ref, f1stat_ref,
              out_ref):
    w = _bn_relu(w_ref[...], wstat_ref, float(_P))
    w3 = w.reshape(_RBF, _K, 128)
    m = jnp.max(w3, axis=1, keepdims=True)
    e = jnp.exp(w3 - m)
    att = e / jnp.sum(e, axis=1, keepdims=True)
    fpc = _bn_relu(z1_ref[...], z1stat_ref, float(_P)).reshape(_RBF, _K, 128)
    pooled = jnp.sum(att * fpc, axis=1)
    npc = _bn_relu(f1_ref[...], f1stat_ref, float(_BS))
    out_ref[...] = npc + pooled


def _blk(c):
    return pl.BlockSpec((_RB, c), lambda i: (i, 0))


def _blkf(c):
    return pl.BlockSpec((_RBF, c), lambda i: (i, 0))


def _full(shape):
    return pl.BlockSpec(shape, lambda i: tuple(0 for _ in shape))


def _stat_spec(c):
    return pl.BlockSpec((8, c), lambda i: (0, 0))


def _stat_shape(c):
    return jax.ShapeDtypeStruct((8, c), jnp.float32)


def _run_dense(gx, nx, gp, fpts, pawt, pab, w0pt, w0at, b0, w1t, b1,
               lwt, lb):
    y, ystat = pl.pallas_call(
        _dk1_body,
        grid=(_NBLK,),
        in_specs=[_blk(16), _blkf(3), _full((10, 32)), _full((1, 32))],
        out_specs=[_blk(32), _stat_spec(32)],
        out_shape=[jax.ShapeDtypeStruct((_P, 32), jnp.float32),
                   _stat_shape(32)],
    )(gx, nx, pawt, pab)

    z0, f0, z0stat, f0stat = pl.pallas_call(
        _dk2_body,
        grid=(_NBLK,),
        in_specs=[_blk(32), _stat_spec(32), _blk(64), _blkf(64),
                  _full((64, 64)), _full((32, 64)), _full((1, 64))],
        out_specs=[_blk(64), _blkf(64), _stat_spec(64), _stat_spec(64)],
        out_shape=[jax.ShapeDtypeStruct((_P, 64), jnp.float32),
                   jax.ShapeDtypeStruct((_BS, 64), jnp.float32),
                   _stat_shape(64), _stat_shape(64)],
    )(y, ystat, gp, fpts, w0pt, w0at, b0)

    z1, f1, z1stat, f1stat = pl.pallas_call(
        _dk4_body,
        grid=(_NBLK,),
        in_specs=[_blk(64), _stat_spec(64), _blkf(64), _stat_spec(64),
                  _full((64, 128)), _full((1, 128))],
        out_specs=[_blk(128), _blkf(128), _stat_spec(128), _stat_spec(128)],
        out_shape=[jax.ShapeDtypeStruct((_P, 128), jnp.float32),
                   jax.ShapeDtypeStruct((_BS, 128), jnp.float32),
                   _stat_shape(128), _stat_shape(128)],
    )(z0, z0stat, f0, f0stat, w1t, b1)

    wr, wstat = pl.pallas_call(
        _dk5_body,
        grid=(_NBLK,),
        in_specs=[_blk(128), _stat_spec(128), _blkf(128), _stat_spec(128),
                  _blk(16), _blkf(3), _full((135, 128)), _full((1, 128))],
        out_specs=[_blk(128), _stat_spec(128)],
        out_shape=[jax.ShapeDtypeStruct((_P, 128), jnp.float32),
                   _stat_shape(128)],
    )(z1, z1stat, f1, f1stat, gx, nx, lwt, lb)

    out = pl.pallas_call(
        _dk6_body,
        grid=(_NBLK,),
        in_specs=[_blk(128), _stat_spec(128), _blk(128), _stat_spec(128),
                  _blkf(128), _stat_spec(128)],
        out_specs=_blkf(128),
        out_shape=jax.ShapeDtypeStruct((_BS, 128), jnp.float32),
    )(wr, wstat, z1, z1stat, f1, f1stat)
    return out


def kernel(xyz, points, pa_w, pa_b, w0, b0, w1, b1, laa_w, laa_b):
    xyz3 = jnp.transpose(xyz, (1, 0, 2))
    start = jax.random.randint(jax.random.key(42), (_B,), 0, _N,
                               dtype=jnp.int32).reshape(_B, 1)
    cents = _run_fps(xyz3, start)
    new_xyz, nbr_g, cents_g = _run_knn(xyz3, cents.reshape(_B, _S, 1))

    gidx = jnp.concatenate([nbr_g.reshape(-1), cents_g.reshape(-1)])
    xyz_t, pts_t = _build_tables(xyz, points)
    gx, gp, fpts = _gather_rows(xyz_t, pts_t, gidx)
    nx = new_xyz.reshape(_BS, 3)

    pawt = pa_w.T
    pab = pa_b.reshape(1, -1)
    w0t = w0.T
    w0pt = w0t[:_D]
    w0at = w0t[_D:]
    b0r = b0.reshape(1, -1)
    w1t = w1.T
    b1r = b1.reshape(1, -1)
    lwt = laa_w.T
    lbr = laa_b.reshape(1, -1)

    out = _run_dense(gx, nx, gp, fpts, pawt, pab, w0pt, w0at, b0r,
                     w1t, b1r, lwt, lbr)

    out1 = jnp.transpose(new_xyz, (0, 2, 1))
    out2 = jnp.transpose(out.reshape(_B, _S, 128), (0, 2, 1))
    return (out1, out2)


# speedup vs baseline: 1.0114x; 1.0114x over previous
import functools

import jax
import jax.numpy as jnp
from jax import lax
from jax.experimental import pallas as pl
from jax.experimental.pallas import tpu as pltpu
from jax.experimental.pallas import tpu_sc as plsc

_B = 8
_N = 4096
_D = 64
_S = 512
_K = 32
_P = _B * _S * _K
_BS = _B * _S
_T = _P + _BS
_RB = 4096
_RBF = 128
_NBLK = _P // _RB
_EPS = 1e-5
_TW = 80


def _fps_body(xyz_ref, start_ref, cent_ref):
    x = xyz_ref[0]
    y = xyz_ref[1]
    z = xyz_ref[2]
    xyz24 = jnp.concatenate([x, y, z], axis=0)
    iota_n = lax.broadcasted_iota(jnp.int32, (_B, _N), 1)
    iota_n24 = lax.broadcasted_iota(jnp.int32, (3 * _B, _N), 1)
    iota_s = lax.broadcasted_iota(jnp.int32, (_B, _S), 1)
    far0 = start_ref[...]
    dist0 = jnp.full((_B, _N), 1e10, dtype=jnp.float32)
    cents0 = jnp.zeros((_B, _S), dtype=jnp.int32)

    def body(i, state):
        distance, far, cents = state
        cents = jnp.where(iota_s == i, far, cents)
        far24 = jnp.concatenate([far, far, far], axis=0)
        csum = jnp.sum(jnp.where(iota_n24 == far24, xyz24, 0.0),
                       axis=1, keepdims=True)
        dx = x - csum[0:_B]
        dy = y - csum[_B:2 * _B]
        dz = z - csum[2 * _B:3 * _B]
        d = (dx * dx + dy * dy) + dz * dz
        distance = jnp.minimum(distance, d)
        mx = jnp.max(distance, axis=1, keepdims=True)
        far = jnp.min(jnp.where(distance == mx, iota_n, _N), axis=1,
                      keepdims=True)
        return distance, far, cents

    _, _, cents = lax.fori_loop(0, _S, body, (dist0, far0, cents0))
    cent_ref[...] = cents


def _run_fps(xyz3, start):
    return pl.pallas_call(
        _fps_body,
        out_shape=jax.ShapeDtypeStruct((_B, _S), jnp.int32),
    )(xyz3, start)


_SB = 128


def _knn_body(xyz_ref, cent_ref, nxyz_ref, nbr_ref, cg_ref):
    b = pl.program_id(0)
    c = cent_ref[0]
    xb = xyz_ref[0, 0]
    yb = xyz_ref[1, 0]
    zb = xyz_ref[2, 0]
    iota_n = lax.broadcasted_iota(jnp.int32, (_SB, _N), 1)
    onehot = iota_n == c
    cx = jnp.sum(jnp.where(onehot, xb, 0.0), axis=1, keepdims=True)
    cy = jnp.sum(jnp.where(onehot, yb, 0.0), axis=1, keepdims=True)
    cz = jnp.sum(jnp.where(onehot, zb, 0.0), axis=1, keepdims=True)
    nxyz_ref[...] = jnp.concatenate([cx, cy, cz], axis=1)[None]
    dx = xb - cx
    dy = yb - cy
    dz = zb - cz
    d2 = (dx * dx + dy * dy) + dz * dz
    iota_k = lax.broadcasted_iota(jnp.int32, (_SB, _K), 1)
    key = (lax.bitcast_convert_type(d2, jnp.int32) & ~0xFFF) | iota_n
    big = jnp.int32(0x7FFFFFFF)

    def sel_body(k, state):
        keyc, sel = state
        m = jnp.min(keyc, axis=1, keepdims=True)
        keyc = jnp.where(keyc == m, big, keyc)
        sel = jnp.where(iota_k == (k - 1), m & 0xFFF, sel)
        return keyc, sel

    sel0 = jnp.zeros((_SB, _K), dtype=jnp.int32)
    _, sel = lax.fori_loop(0, _K + 1, sel_body, (key, sel0))
    base = b * _N
    nbr_ref[...] = (sel + base)[None]
    cg_ref[...] = (c + base)[None]


def _run_knn(xyz3, cents3):
    xyz4 = xyz3.reshape(3, _B, 1, _N)
    grid = (_B, _S // _SB)
    return pl.pallas_call(
        _knn_body,
        grid=grid,
        in_specs=[
            pl.BlockSpec((3, 1, 1, _N), lambda b, s: (0, b, 0, 0)),
            pl.BlockSpec((1, _SB, 1), lambda b, s: (b, s, 0)),
        ],
        out_specs=[
            pl.BlockSpec((1, _SB, 3), lambda b, s: (b, s, 0)),
            pl.BlockSpec((1, _SB, _K), lambda b, s: (b, s, 0)),
            pl.BlockSpec((1, _SB, 1), lambda b, s: (b, s, 0)),
        ],
        out_shape=[
            jax.ShapeDtypeStruct((_B, _S, 3), jnp.float32),
            jax.ShapeDtypeStruct((_B, _S, _K), jnp.int32),
            jax.ShapeDtypeStruct((_B, _S, 1), jnp.int32),
        ],
    )(xyz4, cents3)


def _tp_pts_body(p_ref, o_ref):
    eye = (lax.broadcasted_iota(jnp.int32, (_D, _D), 0)
           == lax.broadcasted_iota(jnp.int32, (_D, _D), 1)).astype(jnp.float32)
    o_ref[0] = lax.dot_general(p_ref[0], eye, (((0,), (0,)), ((), ())),
                               preferred_element_type=jnp.float32)


def _tp_xyz_body(x_ref, o_ref):
    eye = (lax.broadcasted_iota(jnp.int32, (3, 16), 0)
           == lax.broadcasted_iota(jnp.int32, (3, 16), 1)).astype(jnp.float32)
    o_ref[0] = lax.dot_general(x_ref[0], eye, (((0,), (0,)), ((), ())),
                               preferred_element_type=jnp.float32)


_NT = 2048


def _build_tables(xyz, points):
    pts_t = pl.pallas_call(
        _tp_pts_body,
        grid=(_B, _N // _NT),
        in_specs=[pl.BlockSpec((1, _D, _NT), lambda b, j: (b, 0, j))],
        out_specs=pl.BlockSpec((1, _NT, _D), lambda b, j: (b, j, 0)),
        out_shape=jax.ShapeDtypeStruct((_B, _N, _D), jnp.float32),
    )(points)
    xyz_t = pl.pallas_call(
        _tp_xyz_body,
        grid=(_B, _N // _NT),
        in_specs=[pl.BlockSpec((1, 3, _NT), lambda b, j: (b, 0, j))],
        out_specs=pl.BlockSpec((1, _NT, 16), lambda b, j: (b, j, 0)),
        out_shape=jax.ShapeDtypeStruct((_B, _N, 16), jnp.float32),
    )(xyz)
    return xyz_t.reshape(_B * _N, 16), pts_t.reshape(_B * _N, _D)


_NW = 32
_CH = 128
_WPP = _P // _NW
_NCHP = _WPP // _CH
_WPF = _BS // _NW


def _gather_rows(xyz_t, pts_t, gidx):
    mesh = plsc.VectorSubcoreMesh(core_axis_name="c", subcore_axis_name="s")

    @functools.partial(
        pl.kernel,
        mesh=mesh,
        compiler_params=pltpu.CompilerParams(use_tc_tiling_on_sc=False),
        out_type=[
            jax.ShapeDtypeStruct((_P, 16), jnp.float32),
            jax.ShapeDtypeStruct((_P, _D), jnp.float32),
            jax.ShapeDtypeStruct((_BS, _D), jnp.float32),
        ],
        scratch_types=[
            pltpu.VMEM((_WPP + _WPF,), jnp.int32),
            pltpu.VMEM((2, _CH, 16), jnp.float32),
            pltpu.VMEM((2, _CH, _D), jnp.float32),
            pltpu.SemaphoreType.DMA,
            pltpu.SemaphoreType.DMA,
        ],
    )
    def body(xyz_hbm, pts_hbm, gidx_hbm, gx_hbm, gp_hbm, fp_hbm,
             idx_v, xbuf, pbuf, sem1, sem2):
        wid = lax.axis_index("s") * 2 + lax.axis_index("c")
        basep = wid * _WPP
        basef = wid * _WPF
        pltpu.sync_copy(gidx_hbm.at[pl.ds(basep, _WPP)],
                        idx_v.at[pl.ds(0, _WPP)])
        pltpu.sync_copy(gidx_hbm.at[pl.ds(_P + basef, _WPF)],
                        idx_v.at[pl.ds(_WPP, _WPF)])

        def start(j, slot):
            pltpu.async_copy(
                pts_hbm.at[idx_v.at[pl.ds(j * _CH, _CH)]],
                pbuf.at[slot], sem1)
            pltpu.async_copy(
                xyz_hbm.at[idx_v.at[pl.ds(j * _CH, _CH)]],
                xbuf.at[slot], sem2)

        def drain(j, slot):
            pltpu.make_async_copy(
                pts_hbm.at[pl.ds(0, _CH)], pbuf.at[slot], sem1).wait()
            pltpu.make_async_copy(
                xyz_hbm.at[pl.ds(0, _CH)], xbuf.at[slot], sem2).wait()
            pltpu.sync_copy(pbuf.at[slot],
                            gp_hbm.at[pl.ds(basep + j * _CH, _CH)])
            pltpu.sync_copy(xbuf.at[slot],
                            gx_hbm.at[pl.ds(basep + j * _CH, _CH)])

        start(0, 0)

        def chunk2(h, carry):
            start(2 * h + 1, 1)
            drain(2 * h, 0)
            start(2 * h + 2, 0)
            drain(2 * h + 1, 1)
            return carry

        lax.fori_loop(0, _NCHP // 2 - 1, chunk2, 0, unroll=False)
        start(_NCHP - 1, 1)
        drain(_NCHP - 2, 0)
        drain(_NCHP - 1, 1)
        pltpu.async_copy(
            pts_hbm.at[idx_v.at[pl.ds(_WPP, _WPF)]], pbuf.at[0], sem1).wait()
        pltpu.sync_copy(pbuf.at[0], fp_hbm.at[pl.ds(basef, _WPF)])

    return body(xyz_t, pts_t, gidx)


def _rep_mat():
    rows = lax.broadcasted_iota(jnp.int32, (_RB, _RBF), 0) // _K
    cols = lax.broadcasted_iota(jnp.int32, (_RB, _RBF), 1)
    return (rows == cols).astype(jnp.float32)


def _bn_relu(x, stat_ref, count):
    st = stat_ref[...]
    mean = st[0:1, :] / count
    var = st[1:2, :] / count - mean * mean
    return jnp.maximum((x - mean) * lax.rsqrt(var + _EPS), 0.0)


def _acc_stats(stat_ref, y):
    s = jnp.sum(y, axis=0, keepdims=True)
    s2 = jnp.sum(y * y, axis=0, keepdims=True)
    st = jnp.concatenate([s, s2, jnp.zeros((6, y.shape[1]), jnp.float32)],
                         axis=0)

    @pl.when(pl.program_id(0) == 0)
    def _():
        stat_ref[...] = st

    @pl.when(pl.program_id(0) != 0)
    def _():
        stat_ref[...] += st


def _dk1_body(gx_ref, nx_ref, pawt_ref, pab_ref, y_ref, stat_ref):
    gxyz = gx_ref[...][:, 0:3]
    cen = jnp.dot(_rep_mat(), nx_ref[...],
                  preferred_element_type=jnp.float32)
    d = gxyz - cen
    gnorm = d * d
    gdist = jnp.sqrt(jnp.sum(gnorm, axis=1, keepdims=True))
    feat = jnp.concatenate([cen, gxyz, gnorm, gdist], axis=1)
    y = jnp.dot(feat, pawt_ref[...],
                preferred_element_type=jnp.float32) + pab_ref[...]
    y_ref[...] = y
    _acc_stats(stat_ref, y)


def _dk2_body(y_ref, ystat_ref, gp_ref, fp_ref, w0pt_ref, w0at_ref, b0_ref,
              z0_ref, f0_ref, z0stat_ref, f0stat_ref):
    aug = _bn_relu(y_ref[...], ystat_ref, float(_P))
    z0 = (jnp.dot(gp_ref[...], w0pt_ref[...],
                  preferred_element_type=jnp.float32)
          + jnp.dot(aug, w0at_ref[...], preferred_element_type=jnp.float32)
          + b0_ref[...])
    z0_ref[...] = z0
    _acc_stats(z0stat_ref, z0)
    maxo = jnp.max(aug.reshape(_RBF, _K, 32), axis=1)
    f0 = (jnp.dot(fp_ref[...], w0pt_ref[...],
                  preferred_element_type=jnp.float32)
          + jnp.dot(maxo, w0at_ref[...], preferred_element_type=jnp.float32)
          + b0_ref[...])
    f0_ref[...] = f0
    _acc_stats(f0stat_ref, f0)


def _dk4_body(z0_ref, z0stat_ref, f0_ref, f0stat_ref, w1t_ref, b1_ref,
              z1_ref, f1_ref, z1stat_ref, f1stat_ref):
    z = _bn_relu(z0_ref[...], z0stat_ref, float(_P))
    z1 = jnp.dot(z, w1t_ref[...],
                 preferred_element_type=jnp.float32) + b1_ref[...]
    z1_ref[...] = z1
    _acc_stats(z1stat_ref, z1)
    f = _bn_relu(f0_ref[...], f0stat_ref, float(_BS))
    f1 = jnp.dot(f, w1t_ref[...],
                 preferred_element_type=jnp.float32) + b1_ref[...]
    f1_ref[...] = f1
    _acc_stats(f1stat_ref, f1)


def _dk5_body(z1_ref, z1stat_ref, f1_ref, f1stat_ref, gx_ref, nx_ref,
              lwt_ref, lb_ref, w_ref, wstat_ref):
    fpc = _bn_relu(z1_ref[...], z1stat_ref, float(_P))
    npc = _bn_relu(f1_ref[...], f1stat_ref, float(_BS))
    rep = _rep_mat()
    npc_rep = jnp.dot(rep, npc, preferred_element_type=jnp.float32)
    delta = fpc - npc_rep
    gxyz = gx_ref[...][:, 0:3]
    cen = jnp.dot(rep, nx_ref[...], preferred_element_type=jnp.float32)
    d = gxyz - cen
    gdist = jnp.sqrt(jnp.sum(d * d, axis=1, keepdims=True))
    feat = jnp.concatenate([cen, gxyz, gdist, delta], axis=1)
    w = jnp.dot(feat, lwt_ref[...],
                preferred_element_type=jnp.float32) + lb_ref[...]
    w_ref[...] = w
    _acc_stats(wstat_ref, w)


def _dk6_body(w_ref, wstat_ref, z1_ref, z1stat_ref, f1_ref, f1stat_ref,
              out_ref):
    w = _bn_relu(w_ref[...], wstat_ref, float(_P))
    w3 = w.reshape(_RBF, _K, 128)
    m = jnp.max(w3, axis=1, keepdims=True)
    e = jnp.exp(w3 - m)
    att = e / jnp.sum(e, axis=1, keepdims=True)
    fpc = _bn_relu(z1_ref[...], z1stat_ref, float(_P)).reshape(_RBF, _K, 128)
    pooled = jnp.sum(att * fpc, axis=1)
    npc = _bn_relu(f1_ref[...], f1stat_ref, float(_BS))
    out_ref[...] = npc + pooled


def _blk(c):
    return pl.BlockSpec((_RB, c), lambda i: (i, 0))


def _blkf(c):
    return pl.BlockSpec((_RBF, c), lambda i: (i, 0))


def _full(shape):
    return pl.BlockSpec(shape, lambda i: tuple(0 for _ in shape))


def _stat_spec(c):
    return pl.BlockSpec((8, c), lambda i: (0, 0))


def _stat_shape(c):
    return jax.ShapeDtypeStruct((8, c), jnp.float32)


def _run_dense(gx, nx, gp, fpts, pawt, pab, w0pt, w0at, b0, w1t, b1,
               lwt, lb):
    y, ystat = pl.pallas_call(
        _dk1_body,
        grid=(_NBLK,),
        in_specs=[_blk(16), _blkf(3), _full((10, 32)), _full((1, 32))],
        out_specs=[_blk(32), _stat_spec(32)],
        out_shape=[jax.ShapeDtypeStruct((_P, 32), jnp.float32),
                   _stat_shape(32)],
    )(gx, nx, pawt, pab)

    z0, f0, z0stat, f0stat = pl.pallas_call(
        _dk2_body,
        grid=(_NBLK,),
        in_specs=[_blk(32), _stat_spec(32), _blk(64), _blkf(64),
                  _full((64, 64)), _full((32, 64)), _full((1, 64))],
        out_specs=[_blk(64), _blkf(64), _stat_spec(64), _stat_spec(64)],
        out_shape=[jax.ShapeDtypeStruct((_P, 64), jnp.float32),
                   jax.ShapeDtypeStruct((_BS, 64), jnp.float32),
                   _stat_shape(64), _stat_shape(64)],
    )(y, ystat, gp, fpts, w0pt, w0at, b0)

    z1, f1, z1stat, f1stat = pl.pallas_call(
        _dk4_body,
        grid=(_NBLK,),
        in_specs=[_blk(64), _stat_spec(64), _blkf(64), _stat_spec(64),
                  _full((64, 128)), _full((1, 128))],
        out_specs=[_blk(128), _blkf(128), _stat_spec(128), _stat_spec(128)],
        out_shape=[jax.ShapeDtypeStruct((_P, 128), jnp.float32),
                   jax.ShapeDtypeStruct((_BS, 128), jnp.float32),
                   _stat_shape(128), _stat_shape(128)],
    )(z0, z0stat, f0, f0stat, w1t, b1)

    wr, wstat = pl.pallas_call(
        _dk5_body,
        grid=(_NBLK,),
        in_specs=[_blk(128), _stat_spec(128), _blkf(128), _stat_spec(128),
                  _blk(16), _blkf(3), _full((135, 128)), _full((1, 128))],
        out_specs=[_blk(128), _stat_spec(128)],
        out_shape=[jax.ShapeDtypeStruct((_P, 128), jnp.float32),
                   _stat_shape(128)],
    )(z1, z1stat, f1, f1stat, gx, nx, lwt, lb)

    out = pl.pallas_call(
        _dk6_body,
        grid=(_NBLK,),
        in_specs=[_blk(128), _stat_spec(128), _blk(128), _stat_spec(128),
                  _blkf(128), _stat_spec(128)],
        out_specs=_blkf(128),
        out_shape=jax.ShapeDtypeStruct((_BS, 128), jnp.float32),
    )(wr, wstat, z1, z1stat, f1, f1stat)
    return out


def kernel(xyz, points, pa_w, pa_b, w0, b0, w1, b1, laa_w, laa_b):
    xyz3 = jnp.transpose(xyz, (1, 0, 2))
    start = jax.random.randint(jax.random.key(42), (_B,), 0, _N,
                               dtype=jnp.int32).reshape(_B, 1)
    cents = _run_fps(xyz3, start)
    new_xyz, nbr_g, cents_g = _run_knn(xyz3, cents.reshape(_B, _S, 1))

    gidx = jnp.concatenate([nbr_g.reshape(-1), cents_g.reshape(-1)])
    xyz_t, pts_t = _build_tables(xyz, points)
    gx, gp, fpts = _gather_rows(xyz_t, pts_t, gidx)
    nx = new_xyz.reshape(_BS, 3)

    pawt = pa_w.T
    pab = pa_b.reshape(1, -1)
    w0t = w0.T
    w0pt = w0t[:_D]
    w0at = w0t[_D:]
    b0r = b0.reshape(1, -1)
    w1t = w1.T
    b1r = b1.reshape(1, -1)
    lwt = laa_w.T
    lbr = laa_b.reshape(1, -1)

    out = _run_dense(gx, nx, gp, fpts, pawt, pab, w0pt, w0at, b0r,
                     w1t, b1r, lwt, lbr)

    out1 = jnp.transpose(new_xyz, (0, 2, 1))
    out2 = jnp.transpose(out.reshape(_B, _S, 128), (0, 2, 1))
    return (out1, out2)


# speedup vs baseline: 1.0380x; 1.0263x over previous
import functools

import jax
import jax.numpy as jnp
from jax import lax
from jax.experimental import pallas as pl
from jax.experimental.pallas import tpu as pltpu
from jax.experimental.pallas import tpu_sc as plsc

_B = 8
_N = 4096
_D = 64
_S = 512
_K = 32
_P = _B * _S * _K
_BS = _B * _S
_T = _P + _BS
_RB = 4096
_RBF = 128
_NBLK = _P // _RB
_EPS = 1e-5
_TW = 80


def _fps_body(xyz_ref, start_ref, cent_ref):
    x = xyz_ref[0]
    y = xyz_ref[1]
    z = xyz_ref[2]
    xyz24 = jnp.concatenate([x, y, z], axis=0)
    iota_n = lax.broadcasted_iota(jnp.int32, (_B, _N), 1)
    iota_n24 = lax.broadcasted_iota(jnp.int32, (3 * _B, _N), 1)
    iota_s = lax.broadcasted_iota(jnp.int32, (_B, _S), 1)
    far0 = start_ref[...]
    dist0 = jnp.full((_B, _N), 1e10, dtype=jnp.float32)
    cents0 = jnp.zeros((_B, _S), dtype=jnp.int32)

    def body(i, state):
        distance, far, cents = state
        cents = jnp.where(iota_s == i, far, cents)
        far24 = jnp.concatenate([far, far, far], axis=0)
        csum = jnp.sum(jnp.where(iota_n24 == far24, xyz24, 0.0),
                       axis=1, keepdims=True)
        dx = x - csum[0:_B]
        dy = y - csum[_B:2 * _B]
        dz = z - csum[2 * _B:3 * _B]
        d = (dx * dx + dy * dy) + dz * dz
        distance = jnp.minimum(distance, d)
        mx = jnp.max(distance, axis=1, keepdims=True)
        far = jnp.min(jnp.where(distance == mx, iota_n, _N), axis=1,
                      keepdims=True)
        return distance, far, cents

    _, _, cents = lax.fori_loop(0, _S, body, (dist0, far0, cents0))
    cent_ref[...] = cents


def _run_fps(xyz3, start):
    return pl.pallas_call(
        _fps_body,
        out_shape=jax.ShapeDtypeStruct((_B, _S), jnp.int32),
    )(xyz3, start)


_SB = 512


def _knn_body(xyz_ref, cent_ref, nxyz_ref, nbr_ref, cg_ref):
    b = pl.program_id(0)
    c = cent_ref[0]
    xb = xyz_ref[0, 0]
    yb = xyz_ref[1, 0]
    zb = xyz_ref[2, 0]
    iota_n = lax.broadcasted_iota(jnp.int32, (_SB, _N), 1)
    onehot = iota_n == c
    cx = jnp.sum(jnp.where(onehot, xb, 0.0), axis=1, keepdims=True)
    cy = jnp.sum(jnp.where(onehot, yb, 0.0), axis=1, keepdims=True)
    cz = jnp.sum(jnp.where(onehot, zb, 0.0), axis=1, keepdims=True)
    nxyz_ref[...] = jnp.concatenate([cx, cy, cz], axis=1)[None]
    dx = xb - cx
    dy = yb - cy
    dz = zb - cz
    d2 = (dx * dx + dy * dy) + dz * dz
    iota_k = lax.broadcasted_iota(jnp.int32, (_SB, _K), 1)
    key = (lax.bitcast_convert_type(d2, jnp.int32) & ~0xFFF) | iota_n
    big = jnp.int32(0x7FFFFFFF)

    def sel_body(k, state):
        keyc, sel = state
        m = jnp.min(keyc, axis=1, keepdims=True)
        keyc = jnp.where(keyc == m, big, keyc)
        sel = jnp.where(iota_k == (k - 1), m & 0xFFF, sel)
        return keyc, sel

    sel0 = jnp.zeros((_SB, _K), dtype=jnp.int32)
    _, sel = lax.fori_loop(0, _K + 1, sel_body, (key, sel0))
    base = b * _N
    nbr_ref[...] = (sel + base)[None]
    cg_ref[...] = (c + base)[None]


def _run_knn(xyz3, cents3):
    xyz4 = xyz3.reshape(3, _B, 1, _N)
    grid = (_B, _S // _SB)
    return pl.pallas_call(
        _knn_body,
        grid=grid,
        in_specs=[
            pl.BlockSpec((3, 1, 1, _N), lambda b, s: (0, b, 0, 0)),
            pl.BlockSpec((1, _SB, 1), lambda b, s: (b, s, 0)),
        ],
        out_specs=[
            pl.BlockSpec((1, _SB, 3), lambda b, s: (b, s, 0)),
            pl.BlockSpec((1, _SB, _K), lambda b, s: (b, s, 0)),
            pl.BlockSpec((1, _SB, 1), lambda b, s: (b, s, 0)),
        ],
        out_shape=[
            jax.ShapeDtypeStruct((_B, _S, 3), jnp.float32),
            jax.ShapeDtypeStruct((_B, _S, _K), jnp.int32),
            jax.ShapeDtypeStruct((_B, _S, 1), jnp.int32),
        ],
    )(xyz4, cents3)


def _tp_pts_body(p_ref, o_ref):
    eye = (lax.broadcasted_iota(jnp.int32, (_D, _D), 0)
           == lax.broadcasted_iota(jnp.int32, (_D, _D), 1)).astype(jnp.float32)
    o_ref[0] = lax.dot_general(p_ref[0], eye, (((0,), (0,)), ((), ())),
                               preferred_element_type=jnp.float32)


def _tp_xyz_body(x_ref, o_ref):
    eye = (lax.broadcasted_iota(jnp.int32, (3, 16), 0)
           == lax.broadcasted_iota(jnp.int32, (3, 16), 1)).astype(jnp.float32)
    o_ref[0] = lax.dot_general(x_ref[0], eye, (((0,), (0,)), ((), ())),
                               preferred_element_type=jnp.float32)


_NT = 2048


def _build_tables(xyz, points):
    pts_t = pl.pallas_call(
        _tp_pts_body,
        grid=(_B, _N // _NT),
        in_specs=[pl.BlockSpec((1, _D, _NT), lambda b, j: (b, 0, j))],
        out_specs=pl.BlockSpec((1, _NT, _D), lambda b, j: (b, j, 0)),
        out_shape=jax.ShapeDtypeStruct((_B, _N, _D), jnp.float32),
    )(points)
    xyz_t = pl.pallas_call(
        _tp_xyz_body,
        grid=(_B, _N // _NT),
        in_specs=[pl.BlockSpec((1, 3, _NT), lambda b, j: (b, 0, j))],
        out_specs=pl.BlockSpec((1, _NT, 16), lambda b, j: (b, j, 0)),
        out_shape=jax.ShapeDtypeStruct((_B, _N, 16), jnp.float32),
    )(xyz)
    return xyz_t.reshape(_B * _N, 16), pts_t.reshape(_B * _N, _D)


_NW = 32
_CH = 128
_WPP = _P // _NW
_NCHP = _WPP // _CH
_WPF = _BS // _NW


def _gather_rows(xyz_t, pts_t, gidx):
    mesh = plsc.VectorSubcoreMesh(core_axis_name="c", subcore_axis_name="s")

    @functools.partial(
        pl.kernel,
        mesh=mesh,
        compiler_params=pltpu.CompilerParams(use_tc_tiling_on_sc=False),
        out_type=[
            jax.ShapeDtypeStruct((_P, 16), jnp.float32),
            jax.ShapeDtypeStruct((_P, _D), jnp.float32),
            jax.ShapeDtypeStruct((_BS, _D), jnp.float32),
        ],
        scratch_types=[
            pltpu.VMEM((_WPP + _WPF,), jnp.int32),
            pltpu.VMEM((2, _CH, 16), jnp.float32),
            pltpu.VMEM((2, _CH, _D), jnp.float32),
            pltpu.SemaphoreType.DMA,
            pltpu.SemaphoreType.DMA,
        ],
    )
    def body(xyz_hbm, pts_hbm, gidx_hbm, gx_hbm, gp_hbm, fp_hbm,
             idx_v, xbuf, pbuf, sem1, sem2):
        wid = lax.axis_index("s") * 2 + lax.axis_index("c")
        basep = wid * _WPP
        basef = wid * _WPF
        pltpu.sync_copy(gidx_hbm.at[pl.ds(basep, _WPP)],
                        idx_v.at[pl.ds(0, _WPP)])
        pltpu.sync_copy(gidx_hbm.at[pl.ds(_P + basef, _WPF)],
                        idx_v.at[pl.ds(_WPP, _WPF)])

        def start(j, slot):
            pltpu.async_copy(
                pts_hbm.at[idx_v.at[pl.ds(j * _CH, _CH)]],
                pbuf.at[slot], sem1)
            pltpu.async_copy(
                xyz_hbm.at[idx_v.at[pl.ds(j * _CH, _CH)]],
                xbuf.at[slot], sem2)

        def drain(j, slot):
            pltpu.make_async_copy(
                pts_hbm.at[pl.ds(0, _CH)], pbuf.at[slot], sem1).wait()
            pltpu.make_async_copy(
                xyz_hbm.at[pl.ds(0, _CH)], xbuf.at[slot], sem2).wait()
            pltpu.sync_copy(pbuf.at[slot],
                            gp_hbm.at[pl.ds(basep + j * _CH, _CH)])
            pltpu.sync_copy(xbuf.at[slot],
                            gx_hbm.at[pl.ds(basep + j * _CH, _CH)])

        start(0, 0)

        def chunk2(h, carry):
            start(2 * h + 1, 1)
            drain(2 * h, 0)
            start(2 * h + 2, 0)
            drain(2 * h + 1, 1)
            return carry

        lax.fori_loop(0, _NCHP // 2 - 1, chunk2, 0, unroll=False)
        start(_NCHP - 1, 1)
        drain(_NCHP - 2, 0)
        drain(_NCHP - 1, 1)
        pltpu.async_copy(
            pts_hbm.at[idx_v.at[pl.ds(_WPP, _WPF)]], pbuf.at[0], sem1).wait()
        pltpu.sync_copy(pbuf.at[0], fp_hbm.at[pl.ds(basef, _WPF)])

    return body(xyz_t, pts_t, gidx)


def _rep_mat():
    rows = lax.broadcasted_iota(jnp.int32, (_RB, _RBF), 0) // _K
    cols = lax.broadcasted_iota(jnp.int32, (_RB, _RBF), 1)
    return (rows == cols).astype(jnp.float32)


def _bn_relu(x, stat_ref, count):
    st = stat_ref[...]
    mean = st[0:1, :] / count
    var = st[1:2, :] / count - mean * mean
    return jnp.maximum((x - mean) * lax.rsqrt(var + _EPS), 0.0)


def _acc_stats(stat_ref, y):
    s = jnp.sum(y, axis=0, keepdims=True)
    s2 = jnp.sum(y * y, axis=0, keepdims=True)
    st = jnp.concatenate([s, s2, jnp.zeros((6, y.shape[1]), jnp.float32)],
                         axis=0)

    @pl.when(pl.program_id(0) == 0)
    def _():
        stat_ref[...] = st

    @pl.when(pl.program_id(0) != 0)
    def _():
        stat_ref[...] += st


def _dk1_body(gx_ref, nx_ref, pawt_ref, pab_ref, y_ref, stat_ref):
    gxyz = gx_ref[...][:, 0:3]
    cen = jnp.dot(_rep_mat(), nx_ref[...],
                  preferred_element_type=jnp.float32)
    d = gxyz - cen
    gnorm = d * d
    gdist = jnp.sqrt(jnp.sum(gnorm, axis=1, keepdims=True))
    feat = jnp.concatenate([cen, gxyz, gnorm, gdist], axis=1)
    y = jnp.dot(feat, pawt_ref[...],
                preferred_element_type=jnp.float32) + pab_ref[...]
    y_ref[...] = y
    _acc_stats(stat_ref, y)


def _dk2_body(y_ref, ystat_ref, gp_ref, fp_ref, w0pt_ref, w0at_ref, b0_ref,
              z0_ref, f0_ref, z0stat_ref, f0stat_ref):
    aug = _bn_relu(y_ref[...], ystat_ref, float(_P))
    z0 = (jnp.dot(gp_ref[...], w0pt_ref[...],
                  preferred_element_type=jnp.float32)
          + jnp.dot(aug, w0at_ref[...], preferred_element_type=jnp.float32)
          + b0_ref[...])
    z0_ref[...] = z0
    _acc_stats(z0stat_ref, z0)
    maxo = jnp.max(aug.reshape(_RBF, _K, 32), axis=1)
    f0 = (jnp.dot(fp_ref[...], w0pt_ref[...],
                  preferred_element_type=jnp.float32)
          + jnp.dot(maxo, w0at_ref[...], preferred_element_type=jnp.float32)
          + b0_ref[...])
    f0_ref[...] = f0
    _acc_stats(f0stat_ref, f0)


def _dk4_body(z0_ref, z0stat_ref, f0_ref, f0stat_ref, w1t_ref, b1_ref,
              z1_ref, f1_ref, z1stat_ref, f1stat_ref):
    z = _bn_relu(z0_ref[...], z0stat_ref, float(_P))
    z1 = jnp.dot(z, w1t_ref[...],
                 preferred_element_type=jnp.float32) + b1_ref[...]
    z1_ref[...] = z1
    _acc_stats(z1stat_ref, z1)
    f = _bn_relu(f0_ref[...], f0stat_ref, float(_BS))
    f1 = jnp.dot(f, w1t_ref[...],
                 preferred_element_type=jnp.float32) + b1_ref[...]
    f1_ref[...] = f1
    _acc_stats(f1stat_ref, f1)


def _dk5_body(z1_ref, z1stat_ref, f1_ref, f1stat_ref, gx_ref, nx_ref,
              lwt_ref, lb_ref, w_ref, wstat_ref):
    fpc = _bn_relu(z1_ref[...], z1stat_ref, float(_P))
    npc = _bn_relu(f1_ref[...], f1stat_ref, float(_BS))
    rep = _rep_mat()
    npc_rep = jnp.dot(rep, npc, preferred_element_type=jnp.float32)
    delta = fpc - npc_rep
    gxyz = gx_ref[...][:, 0:3]
    cen = jnp.dot(rep, nx_ref[...], preferred_element_type=jnp.float32)
    d = gxyz - cen
    gdist = jnp.sqrt(jnp.sum(d * d, axis=1, keepdims=True))
    feat = jnp.concatenate([cen, gxyz, gdist, delta], axis=1)
    w = jnp.dot(feat, lwt_ref[...],
                preferred_element_type=jnp.float32) + lb_ref[...]
    w_ref[...] = w
    _acc_stats(wstat_ref, w)


def _dk6_body(w_ref, wstat_ref, z1_ref, z1stat_ref, f1_ref, f1stat_ref,
              out_ref):
    w = _bn_relu(w_ref[...], wstat_ref, float(_P))
    w3 = w.reshape(_RBF, _K, 128)
    m = jnp.max(w3, axis=1, keepdims=True)
    e = jnp.exp(w3 - m)
    att = e / jnp.sum(e, axis=1, keepdims=True)
    fpc = _bn_relu(z1_ref[...], z1stat_ref, float(_P)).reshape(_RBF, _K, 128)
    pooled = jnp.sum(att * fpc, axis=1)
    npc = _bn_relu(f1_ref[...], f1stat_ref, float(_BS))
    out_ref[...] = npc + pooled


def _blk(c):
    return pl.BlockSpec((_RB, c), lambda i: (i, 0))


def _blkf(c):
    return pl.BlockSpec((_RBF, c), lambda i: (i, 0))


def _full(shape):
    return pl.BlockSpec(shape, lambda i: tuple(0 for _ in shape))


def _stat_spec(c):
    return pl.BlockSpec((8, c), lambda i: (0, 0))


def _stat_shape(c):
    return jax.ShapeDtypeStruct((8, c), jnp.float32)


def _run_dense(gx, nx, gp, fpts, pawt, pab, w0pt, w0at, b0, w1t, b1,
               lwt, lb):
    y, ystat = pl.pallas_call(
        _dk1_body,
        grid=(_NBLK,),
        in_specs=[_blk(16), _blkf(3), _full((10, 32)), _full((1, 32))],
        out_specs=[_blk(32), _stat_spec(32)],
        out_shape=[jax.ShapeDtypeStruct((_P, 32), jnp.float32),
                   _stat_shape(32)],
    )(gx, nx, pawt, pab)

    z0, f0, z0stat, f0stat = pl.pallas_call(
        _dk2_body,
        grid=(_NBLK,),
        in_specs=[_blk(32), _stat_spec(32), _blk(64), _blkf(64),
                  _full((64, 64)), _full((32, 64)), _full((1, 64))],
        out_specs=[_blk(64), _blkf(64), _stat_spec(64), _stat_spec(64)],
        out_shape=[jax.ShapeDtypeStruct((_P, 64), jnp.float32),
                   jax.ShapeDtypeStruct((_BS, 64), jnp.float32),
                   _stat_shape(64), _stat_shape(64)],
    )(y, ystat, gp, fpts, w0pt, w0at, b0)

    z1, f1, z1stat, f1stat = pl.pallas_call(
        _dk4_body,
        grid=(_NBLK,),
        in_specs=[_blk(64), _stat_spec(64), _blkf(64), _stat_spec(64),
                  _full((64, 128)), _full((1, 128))],
        out_specs=[_blk(128), _blkf(128), _stat_spec(128), _stat_spec(128)],
        out_shape=[jax.ShapeDtypeStruct((_P, 128), jnp.float32),
                   jax.ShapeDtypeStruct((_BS, 128), jnp.float32),
                   _stat_shape(128), _stat_shape(128)],
    )(z0, z0stat, f0, f0stat, w1t, b1)

    wr, wstat = pl.pallas_call(
        _dk5_body,
        grid=(_NBLK,),
        in_specs=[_blk(128), _stat_spec(128), _blkf(128), _stat_spec(128),
                  _blk(16), _blkf(3), _full((135, 128)), _full((1, 128))],
        out_specs=[_blk(128), _stat_spec(128)],
        out_shape=[jax.ShapeDtypeStruct((_P, 128), jnp.float32),
                   _stat_shape(128)],
    )(z1, z1stat, f1, f1stat, gx, nx, lwt, lb)

    out = pl.pallas_call(
        _dk6_body,
        grid=(_NBLK,),
        in_specs=[_blk(128), _stat_spec(128), _blk(128), _stat_spec(128),
                  _blkf(128), _stat_spec(128)],
        out_specs=_blkf(128),
        out_shape=jax.ShapeDtypeStruct((_BS, 128), jnp.float32),
    )(wr, wstat, z1, z1stat, f1, f1stat)
    return out


def kernel(xyz, points, pa_w, pa_b, w0, b0, w1, b1, laa_w, laa_b):
    xyz3 = jnp.transpose(xyz, (1, 0, 2))
    start = jax.random.randint(jax.random.key(42), (_B,), 0, _N,
                               dtype=jnp.int32).reshape(_B, 1)
    cents = _run_fps(xyz3, start)
    new_xyz, nbr_g, cents_g = _run_knn(xyz3, cents.reshape(_B, _S, 1))

    gidx = jnp.concatenate([nbr_g.reshape(-1), cents_g.reshape(-1)])
    xyz_t, pts_t = _build_tables(xyz, points)
    gx, gp, fpts = _gather_rows(xyz_t, pts_t, gidx)
    nx = new_xyz.reshape(_BS, 3)

    pawt = pa_w.T
    pab = pa_b.reshape(1, -1)
    w0t = w0.T
    w0pt = w0t[:_D]
    w0at = w0t[_D:]
    b0r = b0.reshape(1, -1)
    w1t = w1.T
    b1r = b1.reshape(1, -1)
    lwt = laa_w.T
    lbr = laa_b.reshape(1, -1)

    out = _run_dense(gx, nx, gp, fpts, pawt, pab, w0pt, w0at, b0r,
                     w1t, b1r, lwt, lbr)

    out1 = jnp.transpose(new_xyz, (0, 2, 1))
    out2 = jnp.transpose(out.reshape(_B, _S, 128), (0, 2, 1))
    return (out1, out2)
